# Initial kernel scaffold; baseline (speedup 1.0000x reference)
import jax
import jax.numpy as jnp
from jax.experimental import pallas as pl


def kernel(x, edge_index, batch, params):
    raise NotImplementedError("write your pallas kernel here")


# speedup vs baseline: 1.0001x; 1.0001x over previous
import jax
import jax.numpy as jnp
from jax.experimental import pallas as pl


def _mlp2(h, p1, p2):
    h = jnp.dot(h, p1['W']) + p1['b']
    h = jax.nn.relu(h)
    return jnp.dot(h, p2['W']) + p2['b']


def _bn(h, p, eps=1e-5):
    mu = jnp.mean(h, axis=0)
    var = jnp.var(h, axis=0)
    return (h - mu) / jnp.sqrt(var + eps) * p['gamma'] + p['beta']


def kernel(x, edge_index, batch, params):
    G = 512
    src, dst = edge_index[0], edge_index[1]
    agg = jnp.zeros_like(x).at[dst].add(x[src])
    h = _mlp2(x + agg, params['nn1_l1'], params['nn1_l2'])
    h = jax.nn.relu(h)
    h = _bn(h, params['bn1'])
    agg = jnp.zeros_like(h).at[dst].add(h[src])
    h = _mlp2(h + agg, params['nn2_l1'], params['nn2_l2'])
    h = jax.nn.relu(h)
    h = _bn(h, params['bn2'])
    hg = jax.ops.segment_sum(h, batch, num_segments=G)
    hg = jax.nn.relu(jnp.dot(hg, params['fc1']['W']) + params['fc1']['b'])
    x0 = jnp.dot(hg, params['fc2']['W']) + params['fc2']['b']
    g = jax.nn.relu(jnp.dot(hg, params['gender_l1']['W']) + params['gender_l1']['b'])
    x1 = jnp.dot(g, params['gender_l2']['W']) + params['gender_l2']['b']
    x2 = jnp.zeros((hg.shape[0], 1), jnp.float32)
    return (jax.nn.log_softmax(x0, axis=-1),
            jax.nn.log_softmax(x1, axis=-1),
            x2)


# speedup vs baseline: 8.9018x; 8.9007x over previous
import functools

import jax
import jax.numpy as jnp
from jax import lax
from jax.experimental import pallas as pl
from jax.experimental.pallas import tpu as pltpu
from jax.experimental.pallas import tpu_sc as plsc

N = 100000
E = 1600000
G = 512
DIM = 32
HALF = 16
LN = 128
K = 8
ROWS = E // LN
ROWS_PAD = 12544
E_PAD = ROWS_PAD * LN
NPAD = 100096
HN = 50048
NJ = 64
ZR = 1044
HNT = HN // 16
EPS = 1e-5


def _sc_mesh():
    return plsc.VectorSubcoreMesh(core_axis_name="c", subcore_axis_name="s")


def _edge_loop(table_hbm, srcm_hbm, dstm_hbm, sidx, didx, dloc, rows, acc,
               gsem, ssem, base, trips, lo):
    hi = lo + HN

    def chunk(g, carry):
        row0 = base + g * K
        pltpu.sync_copy(srcm_hbm.at[pl.ds(row0, K)], sidx)
        pltpu.sync_copy(dstm_hbm.at[pl.ds(row0, K)], didx)
        for j in range(K):
            for u in range(LN // 16):
                d = didx[j, pl.ds(u * 16, 16)]
                inr = (d >= lo) & (d < hi)
                loc = jnp.where(inr, d - lo, HN + (d & (NJ - 1)))
                dloc[j, pl.ds(u * 16, 16)] = loc
        hs = [pltpu.async_copy(table_hbm.at[sidx.at[j]], rows.at[j], gsem)
              for j in range(K)]
        for h in hs:
            h.wait()
        ss = [pltpu.async_copy(rows.at[j], acc.at[dloc.at[j]], ssem, add=True)
              for j in range(K)]
        for h in ss:
            h.wait()
        return carry
    lax.fori_loop(0, trips, chunk, None)


def _zero_zbuf(zbuf):
    def zstep(i, _):
        zbuf[i, :] = jnp.zeros((HALF,), jnp.float32)
        return _
    lax.fori_loop(0, ZR, zstep, None)


def _agg_passes(edge_fn, out_hbm, zbuf, acc, c, s):
    _zero_zbuf(zbuf)
    for p in range(2):
        lo = p * HN
        for t in range(3):
            pltpu.sync_copy(zbuf, acc.at[pl.ds((s * 3 + t) * ZR, ZR)])
        plsc.subcore_barrier()
        edge_fn(lo)
        plsc.subcore_barrier()
        pltpu.sync_copy(acc.at[pl.ds(s * HNT, HNT)],
                        out_hbm.at[c, pl.ds(lo + s * HNT, HNT)])
        plsc.subcore_barrier()


_SC_SCRATCH = None


def _sc_scratch_types():
    return [
        pltpu.VMEM((K, LN), jnp.int32),
        pltpu.VMEM((K, LN), jnp.int32),
        pltpu.VMEM((K, LN), jnp.int32),
        pltpu.VMEM((K, LN, HALF), jnp.float32),
        pltpu.VMEM((ZR, HALF), jnp.float32),
        pltpu.VMEM_SHARED((HN + NJ, HALF), jnp.float32),
        pltpu.SemaphoreType.DMA,
        pltpu.SemaphoreType.DMA,
    ]


def _sc_agg1(xpad, srcm, dstm):
    rpw = ROWS_PAD // 32
    trips = rpw // K

    def body(xpad_hbm, srcm_hbm, dstm_hbm, out_hbm,
             sidx, didx, dloc, rows, zbuf, acc, gsem, ssem):
        c = lax.axis_index("c")
        s = lax.axis_index("s")
        w = c * 16 + s

        def edge_fn(lo):
            _edge_loop(xpad_hbm, srcm_hbm, dstm_hbm, sidx, didx, dloc, rows,
                       acc, gsem, ssem, w * rpw, trips, lo)

        _agg_passes(edge_fn, out_hbm, zbuf, acc, c, s)

    f = pl.kernel(
        body,
        out_type=jax.ShapeDtypeStruct((2, NPAD, HALF), jnp.float32),
        mesh=_sc_mesh(),
        compiler_params=pltpu.CompilerParams(use_tc_tiling_on_sc=False),
        scratch_types=_sc_scratch_types(),
    )
    return f(xpad, srcm, dstm)


def _sc_agg2(u1a, u1b, srcm, dstm):
    rpw = ROWS_PAD // 16
    trips = rpw // K

    def body(u1a_hbm, u1b_hbm, srcm_hbm, dstm_hbm, out_hbm,
             sidx, didx, dloc, rows, zbuf, acc, gsem, ssem):
        c = lax.axis_index("c")
        s = lax.axis_index("s")

        def edge_fn(lo):
            @pl.when(c == 0)
            def _():
                _edge_loop(u1a_hbm, srcm_hbm, dstm_hbm, sidx, didx, dloc,
                           rows, acc, gsem, ssem, s * rpw, trips, lo)

            @pl.when(c == 1)
            def _():
                _edge_loop(u1b_hbm, srcm_hbm, dstm_hbm, sidx, didx, dloc,
                           rows, acc, gsem, ssem, s * rpw, trips, lo)

        _agg_passes(edge_fn, out_hbm, zbuf, acc, c, s)

    f = pl.kernel(
        body,
        out_type=jax.ShapeDtypeStruct((2, NPAD, HALF), jnp.float32),
        mesh=_sc_mesh(),
        compiler_params=pltpu.CompilerParams(use_tc_tiling_on_sc=False),
        scratch_types=_sc_scratch_types(),
    )
    return f(u1a, u1b, srcm, dstm)


RB = 5000
RD = 2000


def _tc_mlp1(xpad, aggpair, w1, b1, w2, b2):
    nb = N // RB

    def body(x_ref, agg_ref, w1_ref, b1_ref, w2_ref, b2_ref,
             u1a_ref, u1b_ref, deg_ref, s1_ref, q1_ref):
        i = pl.program_id(0)
        t = x_ref[...] + agg_ref[0] + agg_ref[1]
        h = jnp.dot(t, w1_ref[...], preferred_element_type=jnp.float32)
        h = jnp.maximum(h + b1_ref[...], 0.0)
        u = jnp.dot(h, w2_ref[...], preferred_element_type=jnp.float32)
        u = jnp.maximum(u + b2_ref[...], 0.0)
        u1a_ref[...] = u[:, :HALF]
        u1b_ref[...] = u[:, HALF:]
        deg_ref[...] = t[:, 6:7]

        @pl.when(i == 0)
        def _():
            s1_ref[...] = jnp.zeros_like(s1_ref)
            q1_ref[...] = jnp.zeros_like(q1_ref)

        s1_ref[...] += jnp.sum(u, axis=0, keepdims=True)
        q1_ref[...] += jnp.sum(u * u, axis=0, keepdims=True)

    return pl.pallas_call(
        body,
        grid=(nb,),
        in_specs=[
            pl.BlockSpec((RB, HALF), lambda i: (i, 0)),
            pl.BlockSpec((2, RB, HALF), lambda i: (0, i, 0)),
            pl.BlockSpec((HALF, DIM), lambda i: (0, 0)),
            pl.BlockSpec((1, DIM), lambda i: (0, 0)),
            pl.BlockSpec((DIM, DIM), lambda i: (0, 0)),
            pl.BlockSpec((1, DIM), lambda i: (0, 0)),
        ],
        out_specs=[
            pl.BlockSpec((RB, HALF), lambda i: (i, 0)),
            pl.BlockSpec((RB, HALF), lambda i: (i, 0)),
            pl.BlockSpec((RB, 1), lambda i: (i, 0)),
            pl.BlockSpec((1, DIM), lambda i: (0, 0)),
            pl.BlockSpec((1, DIM), lambda i: (0, 0)),
        ],
        out_shape=[
            jax.ShapeDtypeStruct((N, HALF), jnp.float32),
            jax.ShapeDtypeStruct((N, HALF), jnp.float32),
            jax.ShapeDtypeStruct((N, 1), jnp.float32),
            jax.ShapeDtypeStruct((1, DIM), jnp.float32),
            jax.ShapeDtypeStruct((1, DIM), jnp.float32),
        ],
    )(xpad, aggpair, w1, b1, w2, b2)


def _tc_mlp2_pool(u1a, u1b, agg2, deg, batch3, s1, q1, gamma1, beta1,
                  w1, b1, w2, b2):
    nb = N // RD

    def body(u1a_ref, u1b_ref, agg_ref, deg_ref, batch_ref, s1_ref, q1_ref,
             gm_ref, bt_ref, w1_ref, b1_ref, w2_ref, b2_ref,
             pooled_ref, s2_ref, q2_ref):
        i = pl.program_id(0)
        mu = s1_ref[...] / N
        var = q1_ref[...] / N - mu * mu
        scale = gm_ref[...] * lax.rsqrt(var + EPS)
        shift = bt_ref[...] - mu * scale
        u1 = jnp.concatenate([u1a_ref[...], u1b_ref[...]], axis=1)
        agg = jnp.concatenate([agg_ref[0], agg_ref[1]], axis=1)
        t2 = (u1 + agg) * scale + deg_ref[...] * shift
        h = jnp.dot(t2, w1_ref[...], preferred_element_type=jnp.float32)
        h = jnp.maximum(h + b1_ref[...], 0.0)
        u2 = jnp.dot(h, w2_ref[...], preferred_element_type=jnp.float32)
        u2 = jnp.maximum(u2 + b2_ref[...], 0.0)

        bvec = batch_ref[0, 0, :]
        onehot = (bvec[:, None] ==
                  lax.broadcasted_iota(jnp.int32, (1, G), 1)).astype(jnp.float32)
        u2aug = jnp.concatenate(
            [u2, jnp.ones((RD, 1), jnp.float32)], axis=1)
        part = lax.dot_general(onehot, u2aug, (((0,), (0,)), ((), ())),
                               preferred_element_type=jnp.float32)

        @pl.when(i == 0)
        def _():
            pooled_ref[...] = jnp.zeros_like(pooled_ref)
            s2_ref[...] = jnp.zeros_like(s2_ref)
            q2_ref[...] = jnp.zeros_like(q2_ref)

        pooled_ref[...] += part
        s2_ref[...] += jnp.sum(u2, axis=0, keepdims=True)
        q2_ref[...] += jnp.sum(u2 * u2, axis=0, keepdims=True)

    return pl.pallas_call(
        body,
        grid=(nb,),
        in_specs=[
            pl.BlockSpec((RD, HALF), lambda i: (i, 0)),
            pl.BlockSpec((RD, HALF), lambda i: (i, 0)),
            pl.BlockSpec((2, RD, HALF), lambda i: (0, i, 0)),
            pl.BlockSpec((RD, 1), lambda i: (i, 0)),
            pl.BlockSpec((1, 1, RD), lambda i: (i, 0, 0)),
            pl.BlockSpec((1, DIM), lambda i: (0, 0)),
            pl.BlockSpec((1, DIM), lambda i: (0, 0)),
            pl.BlockSpec((1, DIM), lambda i: (0, 0)),
            pl.BlockSpec((1, DIM), lambda i: (0, 0)),
            pl.BlockSpec((DIM, DIM), lambda i: (0, 0)),
            pl.BlockSpec((1, DIM), lambda i: (0, 0)),
            pl.BlockSpec((DIM, DIM), lambda i: (0, 0)),
            pl.BlockSpec((1, DIM), lambda i: (0, 0)),
        ],
        out_specs=[
            pl.BlockSpec((G, DIM + 1), lambda i: (0, 0)),
            pl.BlockSpec((1, DIM), lambda i: (0, 0)),
            pl.BlockSpec((1, DIM), lambda i: (0, 0)),
        ],
        out_shape=[
            jax.ShapeDtypeStruct((G, DIM + 1), jnp.float32),
            jax.ShapeDtypeStruct((1, DIM), jnp.float32),
            jax.ShapeDtypeStruct((1, DIM), jnp.float32),
        ],
    )(u1a, u1b, agg2, deg, batch3, s1, q1, gamma1, beta1, w1, b1, w2, b2)


def _tc_heads(pooled, s2, q2, gamma2, beta2,
              fc1w, fc1b, fc2w, fc2b, gw1, gb1, gw2, gb2):
    def _log_softmax(l):
        m = jnp.max(l, axis=1, keepdims=True)
        return l - m - jnp.log(jnp.sum(jnp.exp(l - m), axis=1, keepdims=True))

    def body(pooled_ref, s2_ref, q2_ref, gm_ref, bt_ref,
             fc1w_ref, fc1b_ref, fc2w_ref, fc2b_ref,
             gw1_ref, gb1_ref, gw2_ref, gb2_ref,
             x0_ref, x1_ref, x2_ref):
        mu = s2_ref[...] / N
        var = q2_ref[...] / N - mu * mu
        scale = gm_ref[...] * lax.rsqrt(var + EPS)
        shift = bt_ref[...] - mu * scale
        pu = pooled_ref[:, :DIM]
        cnt = pooled_ref[:, DIM:DIM + 1]
        hg = pu * scale + cnt * shift
        hg = jnp.maximum(
            jnp.dot(hg, fc1w_ref[...], preferred_element_type=jnp.float32)
            + fc1b_ref[...], 0.0)
        l0 = jnp.dot(hg, fc2w_ref[...],
                     preferred_element_type=jnp.float32) + fc2b_ref[...]
        x0_ref[...] = _log_softmax(l0)
        g = jnp.maximum(
            jnp.dot(hg, gw1_ref[...], preferred_element_type=jnp.float32)
            + gb1_ref[...], 0.0)
        l1 = jnp.dot(g, gw2_ref[...],
                     preferred_element_type=jnp.float32) + gb2_ref[...]
        x1_ref[...] = _log_softmax(l1)
        x2_ref[...] = jnp.zeros_like(x2_ref)

    return pl.pallas_call(
        body,
        out_shape=[
            jax.ShapeDtypeStruct((G, 2), jnp.float32),
            jax.ShapeDtypeStruct((G, 2), jnp.float32),
            jax.ShapeDtypeStruct((G, 1), jnp.float32),
        ],
    )(pooled, s2, q2, gamma2, beta2, fc1w, fc1b, fc2w, fc2b, gw1, gb1,
      gw2, gb2)


def _row(v):
    return v.reshape(1, -1)


def kernel(x, edge_index, batch, params):
    src, dst = edge_index[0], edge_index[1]
    pad = jnp.arange(E_PAD - E, dtype=jnp.int32) % 32
    srcm = jnp.concatenate([src, pad]).reshape(ROWS_PAD, LN)
    dstm = jnp.concatenate([dst, N + pad]).reshape(ROWS_PAD, LN)

    xpad = jnp.concatenate(
        [x, jnp.ones((N, 1), jnp.float32), jnp.zeros((N, 9), jnp.float32)],
        axis=1)
    w1p = jnp.zeros((HALF, DIM), jnp.float32).at[:6].set(
        params['nn1_l1']['W'])

    aggpair = _sc_agg1(xpad, srcm, dstm)
    u1a, u1b, deg, s1, q1 = _tc_mlp1(
        xpad, aggpair, w1p, _row(params['nn1_l1']['b']),
        params['nn1_l2']['W'], _row(params['nn1_l2']['b']))
    agg2 = _sc_agg2(u1a, u1b, srcm, dstm)
    batch3 = batch.reshape(N // RD, 1, RD)
    pooled, s2, q2 = _tc_mlp2_pool(
        u1a, u1b, agg2, deg, batch3, s1, q1,
        _row(params['bn1']['gamma']), _row(params['bn1']['beta']),
        params['nn2_l1']['W'], _row(params['nn2_l1']['b']),
        params['nn2_l2']['W'], _row(params['nn2_l2']['b']))
    x0, x1, x2 = _tc_heads(
        pooled, s2, q2,
        _row(params['bn2']['gamma']), _row(params['bn2']['beta']),
        params['fc1']['W'], _row(params['fc1']['b']),
        params['fc2']['W'], _row(params['fc2']['b']),
        params['gender_l1']['W'], _row(params['gender_l1']['b']),
        params['gender_l2']['W'], _row(params['gender_l2']['b']))
    return (x0, x1, x2)


# speedup vs baseline: 11.0426x; 1.2405x over previous
import functools

import jax
import jax.numpy as jnp
from jax import lax
from jax.experimental import pallas as pl
from jax.experimental.pallas import tpu as pltpu
from jax.experimental.pallas import tpu_sc as plsc

N = 100000
E = 1600000
G = 512
DIM = 32
HALF = 16
LN = 128
K = 8
ROWS = E // LN
ROWS_PAD = 12800
E_PAD = ROWS_PAD * LN
NPAD = 100096
HN = 50048
NJ = 64
ZR = 1044
HNT = HN // 16
EPS = 1e-5


def _sc_mesh():
    return plsc.VectorSubcoreMesh(core_axis_name="c", subcore_axis_name="s")


def _edge_loop(table_hbm, srcm_hbm, dstm_hbm, sidx, didx, dloc, rows, acc,
               isem, gsem, ssem, base, trips, lo):
    hi = lo + HN

    def do_idx(g1, b):
        row0 = base + g1 * K
        h1 = pltpu.async_copy(srcm_hbm.at[pl.ds(row0, K)], sidx.at[b], isem)
        h2 = pltpu.async_copy(dstm_hbm.at[pl.ds(row0, K)], didx.at[b], isem)
        h1.wait()
        h2.wait()

    def mask(b):
        for j in range(K):
            for u in range(LN // 16):
                d = didx[b, j, pl.ds(u * 16, 16)]
                inr = (d >= lo) & (d < hi)
                loc = jnp.where(inr, d - lo, HN + (d & (NJ - 1)))
                dloc[b, j, pl.ds(u * 16, 16)] = loc

    def fire_gathers(b):
        for j in range(K):
            pltpu.async_copy(table_hbm.at[sidx.at[b].at[j]],
                             rows.at[b].at[j], gsem)

    def drain_gathers(b):
        for j in range(K):
            pltpu.make_async_copy(table_hbm.at[sidx.at[b].at[j]],
                                  rows.at[b].at[j], gsem).wait()

    def fire_scatters(b):
        for j in range(K):
            pltpu.async_copy(rows.at[b].at[j], acc.at[dloc.at[b].at[j]],
                             ssem, add=True)

    def drain_scatters(b):
        for j in range(K):
            pltpu.make_async_copy(rows.at[b].at[j], acc.at[dloc.at[b].at[j]],
                                  ssem).wait()

    def step(g1, b, nb):
        drain_gathers(nb)
        fire_scatters(nb)
        fire_gathers(b)
        do_idx(g1, nb)
        drain_scatters(nb)
        mask(nb)

    do_idx(0, 0)
    mask(0)
    fire_gathers(0)
    do_idx(1, 1)
    mask(1)

    def pair(p, carry):
        step(2 * p + 2, 1, 0)
        step(2 * p + 3, 0, 1)
        return carry
    lax.fori_loop(0, (trips - 2) // 2, pair, None)

    drain_gathers(0)
    fire_scatters(0)
    fire_gathers(1)
    drain_scatters(0)
    drain_gathers(1)
    fire_scatters(1)
    drain_scatters(1)


def _zero_zbuf(zbuf):
    def zstep(i, _):
        zbuf[i, :] = jnp.zeros((HALF,), jnp.float32)
        return _
    lax.fori_loop(0, ZR, zstep, None)


def _agg_passes(edge_fn, out_hbm, zbuf, acc, c, s):
    _zero_zbuf(zbuf)
    for p in range(2):
        lo = p * HN
        for t in range(3):
            pltpu.sync_copy(zbuf, acc.at[pl.ds((s * 3 + t) * ZR, ZR)])
        plsc.subcore_barrier()
        edge_fn(lo)
        plsc.subcore_barrier()
        pltpu.sync_copy(acc.at[pl.ds(s * HNT, HNT)],
                        out_hbm.at[c, pl.ds(lo + s * HNT, HNT)])
        plsc.subcore_barrier()


_SC_SCRATCH = None


def _sc_scratch_types():
    return [
        pltpu.VMEM((2, K, LN), jnp.int32),
        pltpu.VMEM((2, K, LN), jnp.int32),
        pltpu.VMEM((2, K, LN), jnp.int32),
        pltpu.VMEM((2, K, LN, HALF), jnp.float32),
        pltpu.VMEM((ZR, HALF), jnp.float32),
        pltpu.VMEM_SHARED((HN + NJ, HALF), jnp.float32),
        pltpu.SemaphoreType.DMA,
        pltpu.SemaphoreType.DMA,
        pltpu.SemaphoreType.DMA,
    ]


def _sc_agg1(xpad, srcm, dstm):
    rpw = ROWS_PAD // 32
    trips = rpw // K

    def body(xpad_hbm, srcm_hbm, dstm_hbm, out_hbm,
             sidx, didx, dloc, rows, zbuf, acc, isem, gsem, ssem):
        c = lax.axis_index("c")
        s = lax.axis_index("s")
        w = c * 16 + s

        def edge_fn(lo):
            _edge_loop(xpad_hbm, srcm_hbm, dstm_hbm, sidx, didx, dloc, rows,
                       acc, isem, gsem, ssem, w * rpw, trips, lo)

        _agg_passes(edge_fn, out_hbm, zbuf, acc, c, s)

    f = pl.kernel(
        body,
        out_type=jax.ShapeDtypeStruct((2, NPAD, HALF), jnp.float32),
        mesh=_sc_mesh(),
        compiler_params=pltpu.CompilerParams(use_tc_tiling_on_sc=False),
        scratch_types=_sc_scratch_types(),
    )
    return f(xpad, srcm, dstm)


def _sc_agg2(u1a, u1b, srcm, dstm):
    rpw = ROWS_PAD // 16
    trips = rpw // K

    def body(u1a_hbm, u1b_hbm, srcm_hbm, dstm_hbm, out_hbm,
             sidx, didx, dloc, rows, zbuf, acc, isem, gsem, ssem):
        c = lax.axis_index("c")
        s = lax.axis_index("s")

        def edge_fn(lo):
            @pl.when(c == 0)
            def _():
                _edge_loop(u1a_hbm, srcm_hbm, dstm_hbm, sidx, didx, dloc,
                           rows, acc, isem, gsem, ssem, s * rpw, trips, lo)

            @pl.when(c == 1)
            def _():
                _edge_loop(u1b_hbm, srcm_hbm, dstm_hbm, sidx, didx, dloc,
                           rows, acc, isem, gsem, ssem, s * rpw, trips, lo)

        _agg_passes(edge_fn, out_hbm, zbuf, acc, c, s)

    f = pl.kernel(
        body,
        out_type=jax.ShapeDtypeStruct((2, NPAD, HALF), jnp.float32),
        mesh=_sc_mesh(),
        compiler_params=pltpu.CompilerParams(use_tc_tiling_on_sc=False),
        scratch_types=_sc_scratch_types(),
    )
    return f(u1a, u1b, srcm, dstm)


RB = 5000
RD = 2000


def _tc_mlp1(xpad, aggpair, w1, b1, w2, b2):
    nb = N // RB

    def body(x_ref, agg_ref, w1_ref, b1_ref, w2_ref, b2_ref,
             u1a_ref, u1b_ref, deg_ref, s1_ref, q1_ref):
        i = pl.program_id(0)
        t = x_ref[...] + agg_ref[0] + agg_ref[1]
        h = jnp.dot(t, w1_ref[...], preferred_element_type=jnp.float32)
        h = jnp.maximum(h + b1_ref[...], 0.0)
        u = jnp.dot(h, w2_ref[...], preferred_element_type=jnp.float32)
        u = jnp.maximum(u + b2_ref[...], 0.0)
        u1a_ref[...] = u[:, :HALF]
        u1b_ref[...] = u[:, HALF:]
        deg_ref[...] = t[:, 6:7]

        @pl.when(i == 0)
        def _():
            s1_ref[...] = jnp.zeros_like(s1_ref)
            q1_ref[...] = jnp.zeros_like(q1_ref)

        s1_ref[...] += jnp.sum(u, axis=0, keepdims=True)
        q1_ref[...] += jnp.sum(u * u, axis=0, keepdims=True)

    return pl.pallas_call(
        body,
        grid=(nb,),
        in_specs=[
            pl.BlockSpec((RB, HALF), lambda i: (i, 0)),
            pl.BlockSpec((2, RB, HALF), lambda i: (0, i, 0)),
            pl.BlockSpec((HALF, DIM), lambda i: (0, 0)),
            pl.BlockSpec((1, DIM), lambda i: (0, 0)),
            pl.BlockSpec((DIM, DIM), lambda i: (0, 0)),
            pl.BlockSpec((1, DIM), lambda i: (0, 0)),
        ],
        out_specs=[
            pl.BlockSpec((RB, HALF), lambda i: (i, 0)),
            pl.BlockSpec((RB, HALF), lambda i: (i, 0)),
            pl.BlockSpec((RB, 1), lambda i: (i, 0)),
            pl.BlockSpec((1, DIM), lambda i: (0, 0)),
            pl.BlockSpec((1, DIM), lambda i: (0, 0)),
        ],
        out_shape=[
            jax.ShapeDtypeStruct((N, HALF), jnp.float32),
            jax.ShapeDtypeStruct((N, HALF), jnp.float32),
            jax.ShapeDtypeStruct((N, 1), jnp.float32),
            jax.ShapeDtypeStruct((1, DIM), jnp.float32),
            jax.ShapeDtypeStruct((1, DIM), jnp.float32),
        ],
    )(xpad, aggpair, w1, b1, w2, b2)


def _tc_mlp2_pool(u1a, u1b, agg2, deg, batch3, s1, q1, gamma1, beta1,
                  w1, b1, w2, b2):
    nb = N // RD

    def body(u1a_ref, u1b_ref, agg_ref, deg_ref, batch_ref, s1_ref, q1_ref,
             gm_ref, bt_ref, w1_ref, b1_ref, w2_ref, b2_ref,
             pooled_ref, s2_ref, q2_ref):
        i = pl.program_id(0)
        mu = s1_ref[...] / N
        var = q1_ref[...] / N - mu * mu
        scale = gm_ref[...] * lax.rsqrt(var + EPS)
        shift = bt_ref[...] - mu * scale
        u1 = jnp.concatenate([u1a_ref[...], u1b_ref[...]], axis=1)
        agg = jnp.concatenate([agg_ref[0], agg_ref[1]], axis=1)
        t2 = (u1 + agg) * scale + deg_ref[...] * shift
        h = jnp.dot(t2, w1_ref[...], preferred_element_type=jnp.float32)
        h = jnp.maximum(h + b1_ref[...], 0.0)
        u2 = jnp.dot(h, w2_ref[...], preferred_element_type=jnp.float32)
        u2 = jnp.maximum(u2 + b2_ref[...], 0.0)

        bvec = batch_ref[0, 0, :]
        onehot = (bvec[:, None] ==
                  lax.broadcasted_iota(jnp.int32, (1, G), 1)).astype(jnp.float32)
        u2aug = jnp.concatenate(
            [u2, jnp.ones((RD, 1), jnp.float32)], axis=1)
        part = lax.dot_general(onehot, u2aug, (((0,), (0,)), ((), ())),
                               preferred_element_type=jnp.float32)

        @pl.when(i == 0)
        def _():
            pooled_ref[...] = jnp.zeros_like(pooled_ref)
            s2_ref[...] = jnp.zeros_like(s2_ref)
            q2_ref[...] = jnp.zeros_like(q2_ref)

        pooled_ref[...] += part
        s2_ref[...] += jnp.sum(u2, axis=0, keepdims=True)
        q2_ref[...] += jnp.sum(u2 * u2, axis=0, keepdims=True)

    return pl.pallas_call(
        body,
        grid=(nb,),
        in_specs=[
            pl.BlockSpec((RD, HALF), lambda i: (i, 0)),
            pl.BlockSpec((RD, HALF), lambda i: (i, 0)),
            pl.BlockSpec((2, RD, HALF), lambda i: (0, i, 0)),
            pl.BlockSpec((RD, 1), lambda i: (i, 0)),
            pl.BlockSpec((1, 1, RD), lambda i: (i, 0, 0)),
            pl.BlockSpec((1, DIM), lambda i: (0, 0)),
            pl.BlockSpec((1, DIM), lambda i: (0, 0)),
            pl.BlockSpec((1, DIM), lambda i: (0, 0)),
            pl.BlockSpec((1, DIM), lambda i: (0, 0)),
            pl.BlockSpec((DIM, DIM), lambda i: (0, 0)),
            pl.BlockSpec((1, DIM), lambda i: (0, 0)),
            pl.BlockSpec((DIM, DIM), lambda i: (0, 0)),
            pl.BlockSpec((1, DIM), lambda i: (0, 0)),
        ],
        out_specs=[
            pl.BlockSpec((G, DIM + 1), lambda i: (0, 0)),
            pl.BlockSpec((1, DIM), lambda i: (0, 0)),
            pl.BlockSpec((1, DIM), lambda i: (0, 0)),
        ],
        out_shape=[
            jax.ShapeDtypeStruct((G, DIM + 1), jnp.float32),
            jax.ShapeDtypeStruct((1, DIM), jnp.float32),
            jax.ShapeDtypeStruct((1, DIM), jnp.float32),
        ],
    )(u1a, u1b, agg2, deg, batch3, s1, q1, gamma1, beta1, w1, b1, w2, b2)


def _tc_heads(pooled, s2, q2, gamma2, beta2,
              fc1w, fc1b, fc2w, fc2b, gw1, gb1, gw2, gb2):
    def _log_softmax(l):
        m = jnp.max(l, axis=1, keepdims=True)
        return l - m - jnp.log(jnp.sum(jnp.exp(l - m), axis=1, keepdims=True))

    def body(pooled_ref, s2_ref, q2_ref, gm_ref, bt_ref,
             fc1w_ref, fc1b_ref, fc2w_ref, fc2b_ref,
             gw1_ref, gb1_ref, gw2_ref, gb2_ref,
             x0_ref, x1_ref, x2_ref):
        mu = s2_ref[...] / N
        var = q2_ref[...] / N - mu * mu
        scale = gm_ref[...] * lax.rsqrt(var + EPS)
        shift = bt_ref[...] - mu * scale
        pu = pooled_ref[:, :DIM]
        cnt = pooled_ref[:, DIM:DIM + 1]
        hg = pu * scale + cnt * shift
        hg = jnp.maximum(
            jnp.dot(hg, fc1w_ref[...], preferred_element_type=jnp.float32)
            + fc1b_ref[...], 0.0)
        l0 = jnp.dot(hg, fc2w_ref[...],
                     preferred_element_type=jnp.float32) + fc2b_ref[...]
        x0_ref[...] = _log_softmax(l0)
        g = jnp.maximum(
            jnp.dot(hg, gw1_ref[...], preferred_element_type=jnp.float32)
            + gb1_ref[...], 0.0)
        l1 = jnp.dot(g, gw2_ref[...],
                     preferred_element_type=jnp.float32) + gb2_ref[...]
        x1_ref[...] = _log_softmax(l1)
        x2_ref[...] = jnp.zeros_like(x2_ref)

    return pl.pallas_call(
        body,
        out_shape=[
            jax.ShapeDtypeStruct((G, 2), jnp.float32),
            jax.ShapeDtypeStruct((G, 2), jnp.float32),
            jax.ShapeDtypeStruct((G, 1), jnp.float32),
        ],
    )(pooled, s2, q2, gamma2, beta2, fc1w, fc1b, fc2w, fc2b, gw1, gb1,
      gw2, gb2)


def _row(v):
    return v.reshape(1, -1)


def kernel(x, edge_index, batch, params):
    src, dst = edge_index[0], edge_index[1]
    pad = jnp.arange(E_PAD - E, dtype=jnp.int32) % 32
    srcm = jnp.concatenate([src, pad]).reshape(ROWS_PAD, LN)
    dstm = jnp.concatenate([dst, N + pad]).reshape(ROWS_PAD, LN)

    xpad = jnp.concatenate(
        [x, jnp.ones((N, 1), jnp.float32), jnp.zeros((N, 9), jnp.float32)],
        axis=1)
    w1p = jnp.zeros((HALF, DIM), jnp.float32).at[:6].set(
        params['nn1_l1']['W'])

    aggpair = _sc_agg1(xpad, srcm, dstm)
    u1a, u1b, deg, s1, q1 = _tc_mlp1(
        xpad, aggpair, w1p, _row(params['nn1_l1']['b']),
        params['nn1_l2']['W'], _row(params['nn1_l2']['b']))
    agg2 = _sc_agg2(u1a, u1b, srcm, dstm)
    batch3 = batch.reshape(N // RD, 1, RD)
    pooled, s2, q2 = _tc_mlp2_pool(
        u1a, u1b, agg2, deg, batch3, s1, q1,
        _row(params['bn1']['gamma']), _row(params['bn1']['beta']),
        params['nn2_l1']['W'], _row(params['nn2_l1']['b']),
        params['nn2_l2']['W'], _row(params['nn2_l2']['b']))
    x0, x1, x2 = _tc_heads(
        pooled, s2, q2,
        _row(params['bn2']['gamma']), _row(params['bn2']['beta']),
        params['fc1']['W'], _row(params['fc1']['b']),
        params['fc2']['W'], _row(params['fc2']['b']),
        params['gender_l1']['W'], _row(params['gender_l1']['b']),
        params['gender_l2']['W'], _row(params['gender_l2']['b']))
    return (x0, x1, x2)


# speedup vs baseline: 14.4512x; 1.3087x over previous
import functools

import jax
import jax.numpy as jnp
from jax import lax
from jax.experimental import pallas as pl
from jax.experimental.pallas import tpu as pltpu
from jax.experimental.pallas import tpu_sc as plsc

N = 100000
E = 1600000
G = 512
DIM = 32
HALF = 16
LN = 128
K = 4
ROWS = E // LN
ROWS_PAD = 12800
E_PAD = ROWS_PAD * LN
NPAD = 100096
ZR = 391
NPT = NPAD // 16
EPS = 1e-5


def _sc_mesh():
    return plsc.VectorSubcoreMesh(core_axis_name="c", subcore_axis_name="s")


def _edge_loop(table_hbm, srcm_hbm, dstm_hbm, sidx, didx, rows, acc,
               isem, gsem, ssem, base, trips):
    def do_idx(g1, b):
        row0 = base + g1 * K
        h1 = pltpu.async_copy(srcm_hbm.at[pl.ds(row0, K)], sidx.at[b], isem)
        h2 = pltpu.async_copy(dstm_hbm.at[pl.ds(row0, K)], didx.at[b], isem)
        h1.wait()
        h2.wait()

    def fire_gathers(b):
        for j in range(K):
            pltpu.async_copy(table_hbm.at[sidx.at[b].at[j]],
                             rows.at[b].at[j], gsem)

    def drain_gathers(b):
        for j in range(K):
            pltpu.make_async_copy(table_hbm.at[sidx.at[b].at[j]],
                                  rows.at[b].at[j], gsem).wait()

    def fire_scatters(b):
        for j in range(K):
            pltpu.async_copy(rows.at[b].at[j], acc.at[didx.at[b].at[j]],
                             ssem, add=True)

    def drain_scatters(b):
        for j in range(K):
            pltpu.make_async_copy(rows.at[b].at[j], acc.at[didx.at[b].at[j]],
                                  ssem).wait()

    def step(g1, b, nb):
        drain_gathers(nb)
        fire_scatters(nb)
        fire_gathers(b)
        do_idx(g1, nb)
        drain_scatters(nb)

    do_idx(0, 0)
    fire_gathers(0)
    do_idx(1, 1)

    def pair(p, carry):
        step(2 * p + 2, 1, 0)
        step(2 * p + 3, 0, 1)
        return carry
    lax.fori_loop(0, (trips - 2) // 2, pair, None)

    drain_gathers(0)
    fire_scatters(0)
    fire_gathers(1)
    drain_scatters(0)
    drain_gathers(1)
    fire_scatters(1)
    drain_scatters(1)


def _zero_zbuf(zbuf):
    def zstep(i, _):
        zbuf[i, :] = jnp.zeros((HALF,), jnp.float32)
        return _
    lax.fori_loop(0, ZR, zstep, None)


def _agg_run(edge_fn, out_hbm, zbuf, acc, c, s):
    _zero_zbuf(zbuf)
    for t in range(16):
        pltpu.sync_copy(zbuf, acc.at[pl.ds((s * 16 + t) * ZR, ZR)])
    plsc.subcore_barrier()
    edge_fn()
    plsc.subcore_barrier()
    pltpu.sync_copy(acc.at[pl.ds(s * NPT, NPT)],
                    out_hbm.at[c, pl.ds(s * NPT, NPT)])


def _sc_scratch_types():
    return [
        pltpu.VMEM((2, K, LN), jnp.int32),
        pltpu.VMEM((2, K, LN), jnp.int32),
        pltpu.VMEM((2, K, LN, HALF), jnp.float32),
        pltpu.VMEM((ZR, HALF), jnp.float32),
        pltpu.VMEM_SHARED((NPAD, HALF), jnp.float32),
        pltpu.SemaphoreType.DMA,
        pltpu.SemaphoreType.DMA,
        pltpu.SemaphoreType.DMA,
    ]


def _sc_agg1(xpad, srcm, dstm):
    rpw = ROWS_PAD // 32
    trips = rpw // K

    def body(xpad_hbm, srcm_hbm, dstm_hbm, out_hbm,
             sidx, didx, rows, zbuf, acc, isem, gsem, ssem):
        c = lax.axis_index("c")
        s = lax.axis_index("s")
        w = c * 16 + s

        def edge_fn():
            _edge_loop(xpad_hbm, srcm_hbm, dstm_hbm, sidx, didx, rows,
                       acc, isem, gsem, ssem, w * rpw, trips)

        _agg_run(edge_fn, out_hbm, zbuf, acc, c, s)

    f = pl.kernel(
        body,
        out_type=jax.ShapeDtypeStruct((2, NPAD, HALF), jnp.float32),
        mesh=_sc_mesh(),
        compiler_params=pltpu.CompilerParams(use_tc_tiling_on_sc=False),
        scratch_types=_sc_scratch_types(),
    )
    return f(xpad, srcm, dstm)


def _sc_agg2(u1a, u1b, srcm, dstm):
    rpw = ROWS_PAD // 16
    trips = rpw // K

    def body(u1a_hbm, u1b_hbm, srcm_hbm, dstm_hbm, out_hbm,
             sidx, didx, rows, zbuf, acc, isem, gsem, ssem):
        c = lax.axis_index("c")
        s = lax.axis_index("s")

        def edge_fn():
            @pl.when(c == 0)
            def _():
                _edge_loop(u1a_hbm, srcm_hbm, dstm_hbm, sidx, didx,
                           rows, acc, isem, gsem, ssem, s * rpw, trips)

            @pl.when(c == 1)
            def _():
                _edge_loop(u1b_hbm, srcm_hbm, dstm_hbm, sidx, didx,
                           rows, acc, isem, gsem, ssem, s * rpw, trips)

        _agg_run(edge_fn, out_hbm, zbuf, acc, c, s)

    f = pl.kernel(
        body,
        out_type=jax.ShapeDtypeStruct((2, NPAD, HALF), jnp.float32),
        mesh=_sc_mesh(),
        compiler_params=pltpu.CompilerParams(use_tc_tiling_on_sc=False),
        scratch_types=_sc_scratch_types(),
    )
    return f(u1a, u1b, srcm, dstm)


RB = 5000
RD = 2000


def _tc_mlp1(xpad, aggpair, w1, b1, w2, b2):
    nb = N // RB

    def body(x_ref, agg_ref, w1_ref, b1_ref, w2_ref, b2_ref,
             u1a_ref, u1b_ref, deg_ref, s1_ref, q1_ref):
        i = pl.program_id(0)
        t = x_ref[...] + agg_ref[0] + agg_ref[1]
        h = jnp.dot(t, w1_ref[...], preferred_element_type=jnp.float32)
        h = jnp.maximum(h + b1_ref[...], 0.0)
        u = jnp.dot(h, w2_ref[...], preferred_element_type=jnp.float32)
        u = jnp.maximum(u + b2_ref[...], 0.0)
        u1a_ref[...] = u[:, :HALF]
        u1b_ref[...] = u[:, HALF:]
        deg_ref[...] = t[:, 6:7]

        @pl.when(i == 0)
        def _():
            s1_ref[...] = jnp.zeros_like(s1_ref)
            q1_ref[...] = jnp.zeros_like(q1_ref)

        s1_ref[...] += jnp.sum(u, axis=0, keepdims=True)
        q1_ref[...] += jnp.sum(u * u, axis=0, keepdims=True)

    return pl.pallas_call(
        body,
        grid=(nb,),
        in_specs=[
            pl.BlockSpec((RB, HALF), lambda i: (i, 0)),
            pl.BlockSpec((2, RB, HALF), lambda i: (0, i, 0)),
            pl.BlockSpec((HALF, DIM), lambda i: (0, 0)),
            pl.BlockSpec((1, DIM), lambda i: (0, 0)),
            pl.BlockSpec((DIM, DIM), lambda i: (0, 0)),
            pl.BlockSpec((1, DIM), lambda i: (0, 0)),
        ],
        out_specs=[
            pl.BlockSpec((RB, HALF), lambda i: (i, 0)),
            pl.BlockSpec((RB, HALF), lambda i: (i, 0)),
            pl.BlockSpec((RB, 1), lambda i: (i, 0)),
            pl.BlockSpec((1, DIM), lambda i: (0, 0)),
            pl.BlockSpec((1, DIM), lambda i: (0, 0)),
        ],
        out_shape=[
            jax.ShapeDtypeStruct((N, HALF), jnp.float32),
            jax.ShapeDtypeStruct((N, HALF), jnp.float32),
            jax.ShapeDtypeStruct((N, 1), jnp.float32),
            jax.ShapeDtypeStruct((1, DIM), jnp.float32),
            jax.ShapeDtypeStruct((1, DIM), jnp.float32),
        ],
    )(xpad, aggpair, w1, b1, w2, b2)


def _tc_mlp2_pool(u1a, u1b, agg2, deg, batch3, s1, q1, gamma1, beta1,
                  w1, b1, w2, b2):
    nb = N // RD

    def body(u1a_ref, u1b_ref, agg_ref, deg_ref, batch_ref, s1_ref, q1_ref,
             gm_ref, bt_ref, w1_ref, b1_ref, w2_ref, b2_ref,
             pooled_ref, s2_ref, q2_ref):
        i = pl.program_id(0)
        mu = s1_ref[...] / N
        var = q1_ref[...] / N - mu * mu
        scale = gm_ref[...] * lax.rsqrt(var + EPS)
        shift = bt_ref[...] - mu * scale
        u1 = jnp.concatenate([u1a_ref[...], u1b_ref[...]], axis=1)
        agg = jnp.concatenate([agg_ref[0], agg_ref[1]], axis=1)
        t2 = (u1 + agg) * scale + deg_ref[...] * shift
        h = jnp.dot(t2, w1_ref[...], preferred_element_type=jnp.float32)
        h = jnp.maximum(h + b1_ref[...], 0.0)
        u2 = jnp.dot(h, w2_ref[...], preferred_element_type=jnp.float32)
        u2 = jnp.maximum(u2 + b2_ref[...], 0.0)

        bvec = batch_ref[0, 0, :]
        onehot = (bvec[:, None] ==
                  lax.broadcasted_iota(jnp.int32, (1, G), 1)).astype(jnp.float32)
        u2aug = jnp.concatenate(
            [u2, jnp.ones((RD, 1), jnp.float32)], axis=1)
        part = lax.dot_general(onehot, u2aug, (((0,), (0,)), ((), ())),
                               preferred_element_type=jnp.float32)

        @pl.when(i == 0)
        def _():
            pooled_ref[...] = jnp.zeros_like(pooled_ref)
            s2_ref[...] = jnp.zeros_like(s2_ref)
            q2_ref[...] = jnp.zeros_like(q2_ref)

        pooled_ref[...] += part
        s2_ref[...] += jnp.sum(u2, axis=0, keepdims=True)
        q2_ref[...] += jnp.sum(u2 * u2, axis=0, keepdims=True)

    return pl.pallas_call(
        body,
        grid=(nb,),
        in_specs=[
            pl.BlockSpec((RD, HALF), lambda i: (i, 0)),
            pl.BlockSpec((RD, HALF), lambda i: (i, 0)),
            pl.BlockSpec((2, RD, HALF), lambda i: (0, i, 0)),
            pl.BlockSpec((RD, 1), lambda i: (i, 0)),
            pl.BlockSpec((1, 1, RD), lambda i: (i, 0, 0)),
            pl.BlockSpec((1, DIM), lambda i: (0, 0)),
            pl.BlockSpec((1, DIM), lambda i: (0, 0)),
            pl.BlockSpec((1, DIM), lambda i: (0, 0)),
            pl.BlockSpec((1, DIM), lambda i: (0, 0)),
            pl.BlockSpec((DIM, DIM), lambda i: (0, 0)),
            pl.BlockSpec((1, DIM), lambda i: (0, 0)),
            pl.BlockSpec((DIM, DIM), lambda i: (0, 0)),
            pl.BlockSpec((1, DIM), lambda i: (0, 0)),
        ],
        out_specs=[
            pl.BlockSpec((G, DIM + 1), lambda i: (0, 0)),
            pl.BlockSpec((1, DIM), lambda i: (0, 0)),
            pl.BlockSpec((1, DIM), lambda i: (0, 0)),
        ],
        out_shape=[
            jax.ShapeDtypeStruct((G, DIM + 1), jnp.float32),
            jax.ShapeDtypeStruct((1, DIM), jnp.float32),
            jax.ShapeDtypeStruct((1, DIM), jnp.float32),
        ],
    )(u1a, u1b, agg2, deg, batch3, s1, q1, gamma1, beta1, w1, b1, w2, b2)


def _tc_heads(pooled, s2, q2, gamma2, beta2,
              fc1w, fc1b, fc2w, fc2b, gw1, gb1, gw2, gb2):
    def _log_softmax(l):
        m = jnp.max(l, axis=1, keepdims=True)
        return l - m - jnp.log(jnp.sum(jnp.exp(l - m), axis=1, keepdims=True))

    def body(pooled_ref, s2_ref, q2_ref, gm_ref, bt_ref,
             fc1w_ref, fc1b_ref, fc2w_ref, fc2b_ref,
             gw1_ref, gb1_ref, gw2_ref, gb2_ref,
             x0_ref, x1_ref, x2_ref):
        mu = s2_ref[...] / N
        var = q2_ref[...] / N - mu * mu
        scale = gm_ref[...] * lax.rsqrt(var + EPS)
        shift = bt_ref[...] - mu * scale
        pu = pooled_ref[:, :DIM]
        cnt = pooled_ref[:, DIM:DIM + 1]
        hg = pu * scale + cnt * shift
        hg = jnp.maximum(
            jnp.dot(hg, fc1w_ref[...], preferred_element_type=jnp.float32)
            + fc1b_ref[...], 0.0)
        l0 = jnp.dot(hg, fc2w_ref[...],
                     preferred_element_type=jnp.float32) + fc2b_ref[...]
        x0_ref[...] = _log_softmax(l0)
        g = jnp.maximum(
            jnp.dot(hg, gw1_ref[...], preferred_element_type=jnp.float32)
            + gb1_ref[...], 0.0)
        l1 = jnp.dot(g, gw2_ref[...],
                     preferred_element_type=jnp.float32) + gb2_ref[...]
        x1_ref[...] = _log_softmax(l1)
        x2_ref[...] = jnp.zeros_like(x2_ref)

    return pl.pallas_call(
        body,
        out_shape=[
            jax.ShapeDtypeStruct((G, 2), jnp.float32),
            jax.ShapeDtypeStruct((G, 2), jnp.float32),
            jax.ShapeDtypeStruct((G, 1), jnp.float32),
        ],
    )(pooled, s2, q2, gamma2, beta2, fc1w, fc1b, fc2w, fc2b, gw1, gb1,
      gw2, gb2)


def _row(v):
    return v.reshape(1, -1)


def kernel(x, edge_index, batch, params):
    src, dst = edge_index[0], edge_index[1]
    pad = jnp.arange(E_PAD - E, dtype=jnp.int32) % 32
    srcm = jnp.concatenate([src, pad]).reshape(ROWS_PAD, LN)
    dstm = jnp.concatenate([dst, N + pad]).reshape(ROWS_PAD, LN)

    xpad = jnp.concatenate(
        [x, jnp.ones((N, 1), jnp.float32), jnp.zeros((N, 9), jnp.float32)],
        axis=1)
    w1p = jnp.zeros((HALF, DIM), jnp.float32).at[:6].set(
        params['nn1_l1']['W'])

    aggpair = _sc_agg1(xpad, srcm, dstm)
    u1a, u1b, deg, s1, q1 = _tc_mlp1(
        xpad, aggpair, w1p, _row(params['nn1_l1']['b']),
        params['nn1_l2']['W'], _row(params['nn1_l2']['b']))
    agg2 = _sc_agg2(u1a, u1b, srcm, dstm)
    batch3 = batch.reshape(N // RD, 1, RD)
    pooled, s2, q2 = _tc_mlp2_pool(
        u1a, u1b, agg2, deg, batch3, s1, q1,
        _row(params['bn1']['gamma']), _row(params['bn1']['beta']),
        params['nn2_l1']['W'], _row(params['nn2_l1']['b']),
        params['nn2_l2']['W'], _row(params['nn2_l2']['b']))
    x0, x1, x2 = _tc_heads(
        pooled, s2, q2,
        _row(params['bn2']['gamma']), _row(params['bn2']['beta']),
        params['fc1']['W'], _row(params['fc1']['b']),
        params['fc2']['W'], _row(params['fc2']['b']),
        params['gender_l1']['W'], _row(params['gender_l1']['b']),
        params['gender_l2']['W'], _row(params['gender_l2']['b']))
    return (x0, x1, x2)
